# baseline (speedup 1.0000x reference)
import jax
import jax.numpy as jnp
from jax import lax
from jax.experimental import pallas as pl
from jax.experimental.pallas import tpu as pltpu
from jax.experimental.pallas import tpu_sc as plsc

_NC = 2
_NS = 16
_NW = _NC * _NS
_LANES = 16


def kernel(token_ids, W):
    B, L = token_ids.shape
    dim = W.shape[1]
    rows_per_w = B // _NW

    mesh = plsc.VectorSubcoreMesh(core_axis_name="c", subcore_axis_name="s")

    @pl.kernel(
        out_type=jax.ShapeDtypeStruct((B, L, dim), W.dtype),
        mesh=mesh,
        scratch_types=[
            pltpu.VMEM((L,), jnp.int32),
            pltpu.SemaphoreType.DMA,
            pltpu.SemaphoreType.DMA,
        ],
    )
    def gather_kernel(w_hbm, i_hbm, o_hbm, idx_v, sem, isem):
        wid = lax.axis_index("s") * _NC + lax.axis_index("c")
        base = wid * rows_per_w

        @pl.loop(0, rows_per_w)
        def _(j):
            gr = base + j
            pltpu.async_copy(i_hbm.at[gr], idx_v, isem).wait()

            @pl.loop(0, L // _LANES)
            def _(g):
                v = idx_v.at[pl.ds(g * _LANES, _LANES)][...]
                for k in range(_LANES):
                    tok = v[k]
                    pltpu.async_copy(w_hbm.at[tok], o_hbm.at[gr, g * _LANES + k], sem)

            tail = L % _LANES
            if tail:
                v = idx_v.at[pl.ds(L - _LANES, _LANES)][...]
                for k in range(_LANES - tail, _LANES):
                    tok = v[k]
                    pltpu.async_copy(w_hbm.at[tok], o_hbm.at[gr, L - _LANES + k], sem)

            pltpu.make_async_copy(w_hbm.at[pl.ds(0, L)], o_hbm.at[gr], sem).wait()

    return gather_kernel(W, token_ids)


# speedup vs baseline: 7.1340x; 7.1340x over previous
import jax
import jax.numpy as jnp
from jax import lax
from jax.experimental import pallas as pl
from jax.experimental.pallas import tpu as pltpu
from jax.experimental.pallas import tpu_sc as plsc

_NC = 2
_NS = 16
_NW = _NC * _NS
_LANES = 16

_RE_LINES = 80
_G_CHUNK = 400


def _repack(W):
    V, dim = W.shape
    lines = V // 4
    n_chunks = lines // _RE_LINES
    mesh = plsc.VectorSubcoreMesh(core_axis_name="c", subcore_axis_name="s")

    @pl.kernel(
        out_type=jax.ShapeDtypeStruct((lines, 128), W.dtype),
        mesh=mesh,
        compiler_params=pltpu.CompilerParams(needs_layout_passes=False),
        scratch_types=[
            pltpu.VMEM((4 * _RE_LINES, dim), jnp.float32),
            pltpu.VMEM((_RE_LINES, 128), jnp.float32),
        ],
    )
    def repack_kernel(w_hbm, o_hbm, vin_v, vout_v):
        wid = lax.axis_index("s") * _NC + lax.axis_index("c")
        iters = n_chunks // _NW + 1

        @pl.loop(0, iters)
        def _(i):
            cid = i * _NW + wid

            @pl.when(cid < n_chunks)
            def _():
                pltpu.sync_copy(w_hbm.at[pl.ds(cid * 4 * _RE_LINES, 4 * _RE_LINES)], vin_v)

                @plsc.parallel_loop(0, _RE_LINES)
                def _(m):
                    for s in range(8):
                        vout_v.at[m, pl.ds(s * _LANES, _LANES)][...] = (
                            vin_v.at[4 * m + s // 2, pl.ds((s % 2) * _LANES, _LANES)][...]
                        )

                pltpu.sync_copy(vout_v, o_hbm.at[pl.ds(cid * _RE_LINES, _RE_LINES)])

    return repack_kernel(W)


def _gather_select(w4, idx, B, L, dim):
    n = idx.shape[0]
    b_per_w = n // _NW
    n_chunks = b_per_w // _G_CHUNK
    rows_per_chunk = _G_CHUNK // L
    mesh = plsc.VectorSubcoreMesh(core_axis_name="c", subcore_axis_name="s")

    @pl.kernel(
        out_type=jax.ShapeDtypeStruct((B, L, dim), w4.dtype),
        mesh=mesh,
        compiler_params=pltpu.CompilerParams(needs_layout_passes=False),
        scratch_types=[
            pltpu.VMEM((_G_CHUNK,), jnp.int32),
            pltpu.VMEM((_G_CHUNK,), jnp.int32),
            pltpu.VMEM((_G_CHUNK,), jnp.int32),
            pltpu.VMEM((_G_CHUNK, 128), jnp.float32),
            pltpu.VMEM((_G_CHUNK, dim), jnp.float32),
            pltpu.SemaphoreType.DMA,
        ],
    )
    def gather_kernel(w_hbm, i_hbm, o_hbm, idx_v, q_v, r_v, wide_v, out_v, sem):
        wid = lax.axis_index("s") * _NC + lax.axis_index("c")
        base_tok = wid * b_per_w
        base_row = wid * (b_per_w // L)
        lanes = lax.iota(jnp.int32, _LANES)

        @pl.loop(0, n_chunks)
        def _(j):
            off = base_tok + j * _G_CHUNK
            pltpu.sync_copy(i_hbm.at[pl.ds(off, _G_CHUNK)], idx_v)

            @pl.loop(0, _G_CHUNK, step=_LANES)
            def _(c):
                v = idx_v.at[pl.ds(c, _LANES)][...]
                q_v.at[pl.ds(c, _LANES)][...] = v >> 2
                r_v.at[pl.ds(c, _LANES)][...] = (v & 3) * dim

            pltpu.async_copy(w_hbm.at[q_v], wide_v, sem).wait()

            @plsc.parallel_loop(0, _G_CHUNK, step=_LANES)
            def _(g):
                row16 = g + lanes
                col0 = r_v.at[pl.ds(g, _LANES)][...]
                for k in range(dim):
                    val = plsc.load_gather(wide_v, [row16, col0 + k])
                    plsc.store_scatter(
                        out_v, [row16, jnp.full((_LANES,), k, jnp.int32)], val
                    )

            gr = base_row + rows_per_chunk * j
            for t in range(rows_per_chunk):
                pltpu.sync_copy(out_v.at[pl.ds(t * L, L)], o_hbm.at[gr + t])

    return gather_kernel(w4, idx)


def kernel(token_ids, W):
    B, L = token_ids.shape
    n = B * L
    dim = W.shape[1]

    idx = token_ids.reshape(n)
    w4 = _repack(W)
    return _gather_select(w4, idx, B, L, dim)


# speedup vs baseline: 9.7811x; 1.3711x over previous
import jax
import jax.numpy as jnp
from jax import lax
from jax.experimental import pallas as pl
from jax.experimental.pallas import tpu as pltpu
from jax.experimental.pallas import tpu_sc as plsc

_NC = 2
_NS = 16
_NW = _NC * _NS
_CHUNK = 800
_SPLITS = 4


def _gather_part(W, idx_part):
    n = idx_part.shape[0]
    b_per_w = n // _NW
    n_chunks = b_per_w // _CHUNK
    mesh = plsc.VectorSubcoreMesh(core_axis_name="c", subcore_axis_name="s")

    @pl.kernel(
        out_type=jax.ShapeDtypeStruct((n, W.shape[1]), W.dtype),
        mesh=mesh,
        compiler_params=pltpu.CompilerParams(use_tc_tiling_on_sc=False),
        scratch_types=[
            pltpu.VMEM((_CHUNK,), jnp.int32),
            pltpu.VMEM((_CHUNK, W.shape[1]), jnp.float32),
            pltpu.SemaphoreType.DMA,
        ],
    )
    def gather_kernel(w_hbm, i_hbm, o_hbm, idx_v, rows_v, sem):
        wid = lax.axis_index("s") * _NC + lax.axis_index("c")
        base = wid * b_per_w

        @pl.loop(0, n_chunks)
        def _(j):
            off = base + j * _CHUNK
            pltpu.sync_copy(i_hbm.at[pl.ds(off, _CHUNK)], idx_v)
            pltpu.async_copy(w_hbm.at[idx_v], rows_v, sem).wait()
            pltpu.sync_copy(rows_v, o_hbm.at[pl.ds(off, _CHUNK)])

    return gather_kernel(W, idx_part)


def kernel(token_ids, W):
    B, L = token_ids.shape
    n = B * L
    dim = W.shape[1]
    idx = token_ids.reshape(n)

    part = n // _SPLITS
    outs = [_gather_part(W, idx[p * part:(p + 1) * part]) for p in range(_SPLITS)]
    out = jnp.concatenate(outs, axis=0)
    return out.reshape(B, L, dim)


# speedup vs baseline: 13.7573x; 1.4065x over previous
import jax
import jax.numpy as jnp
from jax import lax
from jax.experimental import pallas as pl
from jax.experimental.pallas import tpu as pltpu
from jax.experimental.pallas import tpu_sc as plsc

_NC = 2
_NS = 16
_NW = _NC * _NS
_CHUNK = 800
_SPLITS = 4


def _gather_part(W, idx_part):
    n = idx_part.shape[0]
    b_per_w = n // _NW
    n_chunks = b_per_w // _CHUNK
    mesh = plsc.VectorSubcoreMesh(core_axis_name="c", subcore_axis_name="s")

    @pl.kernel(
        out_type=jax.ShapeDtypeStruct((n, W.shape[1]), W.dtype),
        mesh=mesh,
        compiler_params=pltpu.CompilerParams(use_tc_tiling_on_sc=False),
        scratch_types=[
            pltpu.VMEM((_CHUNK,), jnp.int32),
            pltpu.VMEM((_CHUNK, W.shape[1]), jnp.float32),
            pltpu.SemaphoreType.DMA,
        ],
    )
    def gather_kernel(w_hbm, i_hbm, o_hbm, idx_v, rows_v, sem):
        wid = lax.axis_index("s") * _NC + lax.axis_index("c")
        base = wid * b_per_w

        @pl.loop(0, n_chunks)
        def _(j):
            off = base + j * _CHUNK
            pltpu.sync_copy(i_hbm.at[pl.ds(off, _CHUNK)], idx_v)
            pltpu.async_copy(w_hbm.at[idx_v], rows_v, sem).wait()
            pltpu.sync_copy(rows_v, o_hbm.at[pl.ds(off, _CHUNK)])

    return gather_kernel(W, idx_part)


def kernel(token_ids, W):
    B, L = token_ids.shape
    n = B * L
    dim = W.shape[1]
    idx = token_ids.reshape(n)

    part = n // _SPLITS
    outs = [
        _gather_part(W, idx[p * part:(p + 1) * part]).reshape(B // _SPLITS, L, dim)
        for p in range(_SPLITS)
    ]
    return jnp.concatenate(outs, axis=0)
